# baseline (speedup 1.0000x reference)
import functools

import jax
import jax.numpy as jnp
from jax import lax
from jax.experimental import pallas as pl
from jax.experimental.pallas import tpu as pltpu
from jax.experimental.pallas import tpu_sc as plsc

D_MODEL = 64
NUM_WORKERS = 32
CHUNK = 128


def _make_gather(n_chunks: int):
    mesh = plsc.VectorSubcoreMesh(core_axis_name="c", subcore_axis_name="s")

    @functools.partial(
        pl.kernel,
        out_type=jax.ShapeDtypeStruct(
            (NUM_WORKERS * n_chunks * CHUNK, D_MODEL), jnp.float32
        ),
        mesh=mesh,
        compiler_params=pltpu.CompilerParams(use_tc_tiling_on_sc=False),
        scratch_types=[
            pltpu.VMEM((n_chunks, CHUNK), jnp.int32),
            pltpu.VMEM((2, CHUNK, D_MODEL), jnp.float32),
            pltpu.SemaphoreType.DMA,
            pltpu.SemaphoreType.DMA,
        ],
    )
    def gather_kernel(table_hbm, idx_hbm, out_hbm, idx_v, rows_v, gsem, osem):
        num_cores = lax.axis_size("c")
        wid = lax.axis_index("s") * num_cores + lax.axis_index("c")
        base = wid * n_chunks * CHUNK

        pltpu.sync_copy(idx_hbm.at[wid], idx_v)

        def start_gather(j, buf):
            return pltpu.async_copy(
                table_hbm.at[idx_v.at[j]], rows_v.at[buf], gsem
            )

        start_gather(0, 0)

        def body(j, _):
            buf = lax.rem(j, 2)
            pltpu.make_async_copy(
                table_hbm.at[idx_v.at[j]], rows_v.at[buf], gsem
            ).wait()

            @pl.when(j + 1 < n_chunks)
            def _():
                start_gather(j + 1, 1 - buf)

            pltpu.sync_copy(
                rows_v.at[buf], out_hbm.at[pl.ds(base + j * CHUNK, CHUNK)]
            )
            return 0

        lax.fori_loop(0, n_chunks, body, 0)

    return gather_kernel


def kernel(ids, emb_weight):
    batch, hist = ids.shape
    total = batch * hist
    n_chunks = total // (NUM_WORKERS * CHUNK)
    assert n_chunks * NUM_WORKERS * CHUNK == total

    idx = ids.reshape(NUM_WORKERS, n_chunks, CHUNK).astype(jnp.int32)
    out = _make_gather(n_chunks)(emb_weight, idx)
    return out.reshape(batch, hist, D_MODEL)


# speedup vs baseline: 1.0757x; 1.0757x over previous
import functools

import jax
import jax.numpy as jnp
from jax import lax
from jax.experimental import pallas as pl
from jax.experimental.pallas import tpu as pltpu
from jax.experimental.pallas import tpu_sc as plsc

D_MODEL = 64
NUM_WORKERS = 32
CHUNK = 128
K = 5


def _make_gather(n_chunks: int):
    assert n_chunks % K == 0
    n_outer = n_chunks // K
    group = K * CHUNK
    mesh = plsc.VectorSubcoreMesh(core_axis_name="c", subcore_axis_name="s")

    @functools.partial(
        pl.kernel,
        out_type=jax.ShapeDtypeStruct(
            (NUM_WORKERS * n_chunks * CHUNK, D_MODEL), jnp.float32
        ),
        mesh=mesh,
        compiler_params=pltpu.CompilerParams(use_tc_tiling_on_sc=False),
        scratch_types=[
            pltpu.VMEM((n_chunks, CHUNK), jnp.int32),
            pltpu.VMEM((2, group, D_MODEL), jnp.float32),
            pltpu.SemaphoreType.DMA,
            pltpu.SemaphoreType.DMA,
        ],
    )
    def gather_kernel(table_hbm, idx_hbm, out_hbm, idx_v, rows_v, gsem, osem):
        num_cores = lax.axis_size("c")
        wid = lax.axis_index("s") * num_cores + lax.axis_index("c")
        base = wid * n_chunks * CHUNK

        pltpu.sync_copy(idx_hbm.at[wid], idx_v)

        def fire_group(g, buf):
            for i in range(K):
                pltpu.async_copy(
                    table_hbm.at[idx_v.at[g * K + i]],
                    rows_v.at[buf, pl.ds(i * CHUNK, CHUNK)],
                    gsem,
                )

        def drain_group(buf):
            pltpu.make_async_copy(
                table_hbm.at[pl.ds(0, group)], rows_v.at[buf], gsem
            ).wait()

        def drain_out(buf, g):
            pltpu.make_async_copy(
                rows_v.at[buf],
                out_hbm.at[pl.ds(base + g * group, group)],
                osem,
            ).wait()

        fire_group(0, 0)

        def body(g, _):
            buf = lax.rem(g, 2)
            drain_group(buf)

            @pl.when(g + 1 < n_outer)
            def _():
                @pl.when(g >= 1)
                def _():
                    drain_out(1 - buf, g - 1)

                fire_group(g + 1, 1 - buf)

            pltpu.async_copy(
                rows_v.at[buf],
                out_hbm.at[pl.ds(base + g * group, group)],
                osem,
            )
            return 0

        lax.fori_loop(0, n_outer, body, 0)

        drain_out(lax.rem(n_outer, 2), n_outer - 2)
        drain_out(lax.rem(n_outer - 1, 2), n_outer - 1)

    return gather_kernel


def kernel(ids, emb_weight):
    batch, hist = ids.shape
    total = batch * hist
    n_chunks = total // (NUM_WORKERS * CHUNK)
    assert n_chunks * NUM_WORKERS * CHUNK == total

    idx = ids.reshape(NUM_WORKERS, n_chunks, CHUNK).astype(jnp.int32)
    out = _make_gather(n_chunks)(emb_weight, idx)
    return out.reshape(batch, hist, D_MODEL)
